# baseline (speedup 1.0000x reference)
import functools

import jax
import jax.numpy as jnp
from jax import lax
from jax.experimental import pallas as pl
from jax.experimental.pallas import tpu as pltpu
from jax.experimental.pallas import tpu_sc as plsc

N = 10000
E = 160000
D = 128

NC = 2
NS = 16
L = 16

EPW = E // NS
GRP = EPW // L
CPAD = EPW + 128 + L

S = 5
HPC = (N * N) // NC
HPS = HPC // S
FWS = HPS // NS
FCH = 25000
RPS = N // NC // S


def _pq_body(h_ref, w_ref, b_ref, pq_ref):
    h = h_ref[...]
    w1 = w_ref[0, :]
    w2 = w_ref[1, :]
    p = jnp.sum(h * w1[None, :], axis=1) + b_ref[0, 0]
    q = jnp.sum(h * w2[None, :], axis=1)
    pq_ref[pl.ds(0, 1), :] = p.reshape(1, N)
    pq_ref[pl.ds(1, 1), :] = q.reshape(1, N)


_pq_call = pl.pallas_call(
    _pq_body,
    out_shape=jax.ShapeDtypeStruct((2, N), jnp.float32),
)


_mesh = plsc.VectorSubcoreMesh(core_axis_name="c", subcore_axis_name="s")


@functools.partial(
    pl.kernel,
    out_type=jax.ShapeDtypeStruct((N * N,), jnp.float32),
    mesh=_mesh,
    scratch_types=[
        pltpu.VMEM((FCH,), jnp.float32),
        pltpu.VMEM((EPW,), jnp.int32),
        pltpu.VMEM((EPW,), jnp.int32),
        pltpu.VMEM((EPW,), jnp.float32),
        pltpu.VMEM((N,), jnp.float32),
        pltpu.VMEM((N,), jnp.float32),
        pltpu.VMEM((L,), jnp.float32),
        pltpu.VMEM((CPAD,), jnp.int32),
        pltpu.VMEM((CPAD,), jnp.float32),
        pltpu.VMEM((CPAD,), jnp.int32),
        pltpu.VMEM((CPAD,), jnp.float32),
        pltpu.SemaphoreType.DMA,
        pltpu.SemaphoreType.DMA,
    ],
    compiler_params=pltpu.CompilerParams(needs_layout_passes=False),
)
def _sc_kernel(pq_hbm, src_hbm, dst_hbm, w_hbm, c_hbm, out_hbm,
               fill_v, src_v, dst_v, w_v, p_v, q_v, c_v,
               cidx0, cval0, cidx1, cval1, sem, fsem):
    cid = lax.axis_index("c")
    sid = lax.axis_index("s")

    neg = jnp.full((L,), -jnp.inf, jnp.float32)

    def _init_fill(i, carry):
        fill_v[pl.ds(i * L, L)] = neg
        return carry
    lax.fori_loop(0, FCH // L, _init_fill, 0)
    fill_v[pl.ds(FCH - L, L)] = neg

    def _stage_base(s):
        return cid * HPC + s * HPS + sid * FWS

    for s in range(S):
        sbase = _stage_base(s)

        def _fill_start(k, carry, sbase=sbase):
            pltpu.async_copy(fill_v, out_hbm.at[pl.ds(sbase + k * FCH, FCH)],
                             fsem)
            return carry
        lax.fori_loop(0, FWS // FCH, _fill_start, 0)

    eoff = sid * EPW
    pltpu.sync_copy(src_hbm.at[pl.ds(eoff, EPW)], src_v)
    pltpu.sync_copy(dst_hbm.at[pl.ds(eoff, EPW)], dst_v)
    pltpu.sync_copy(w_hbm.at[pl.ds(eoff, EPW)], w_v)
    pltpu.sync_copy(pq_hbm.at[0], p_v)
    pltpu.sync_copy(pq_hbm.at[1], q_v)
    pltpu.sync_copy(c_hbm, c_v)

    c_vec = c_v[...]
    n_vec = jnp.full((L,), N, jnp.int32)
    iota = lax.iota(jnp.int32, L)
    np1_vec = jnp.full((L,), N + 1, jnp.int32)
    bufs = ((cidx0, cval0), (cidx1, cval1))
    stage_rows = []

    def _scatter_drain(cidx_v, cval_v, nrows):
        def _scatter_wait(j, carry):
            pltpu.make_async_copy(cval_v.at[pl.ds(j * 128, 128)],
                                  out_hbm.at[cidx_v.at[pl.ds(j * 128, 128)]],
                                  sem).wait()
            return carry
        lax.fori_loop(0, nrows, _scatter_wait, 0)

    for s in range(S):
        cidx_v, cval_v = bufs[s % 2]
        if s >= 2:
            _scatter_drain(cidx_v, cval_v, stage_rows[s - 2])
        st_lo = cid * (N // NC) + s * RPS
        lo_v = jnp.broadcast_to(st_lo, (L,)).astype(jnp.int32)
        hi_v = lo_v + RPS

        def _compute(j, cnt, lo_v=lo_v, hi_v=hi_v,
                     cidx_v=cidx_v, cval_v=cval_v):
            e0 = j * L
            sv = src_v[pl.ds(e0, L)]
            dv = dst_v[pl.ds(e0, L)]
            wv = w_v[pl.ds(e0, L)]
            pv = plsc.load_gather(p_v, [sv])
            qv = plsc.load_gather(q_v, [dv])
            sc = pv + qv + wv * c_vec
            valid = (sv >= lo_v) & (sv < hi_v) & (sv != dv)
            plsc.store_compressed(cidx_v.at[pl.ds(cnt, L)], sv * n_vec + dv,
                                  mask=valid)
            plsc.store_compressed(cval_v.at[pl.ds(cnt, L)], sc, mask=valid)
            return cnt + jnp.sum(valid.astype(jnp.int32))
        cnt = lax.fori_loop(0, GRP, _compute, jnp.int32(0))

        for t in range(128 // L):
            cidx_v[pl.ds(cnt + t * L, L)] = (lo_v + t * L + iota) * np1_vec
            cval_v[pl.ds(cnt + t * L, L)] = neg
        nrows = (cnt + 127) // 128
        stage_rows.append(nrows)

        sbase = _stage_base(s)

        def _fill_wait(k, carry, sbase=sbase):
            pltpu.make_async_copy(
                fill_v, out_hbm.at[pl.ds(sbase + k * FCH, FCH)], fsem).wait()
            return carry
        lax.fori_loop(0, FWS // FCH, _fill_wait, 0)

        plsc.subcore_barrier()

        def _scatter_start(j, carry, cidx_v=cidx_v, cval_v=cval_v):
            pltpu.async_copy(cval_v.at[pl.ds(j * 128, 128)],
                             out_hbm.at[cidx_v.at[pl.ds(j * 128, 128)]], sem)
            return carry
        lax.fori_loop(0, nrows, _scatter_start, 0)

    for s in range(S - 2, S):
        cidx_v, cval_v = bufs[s % 2]
        _scatter_drain(cidx_v, cval_v, stage_rows[s])


def kernel(h, edge_index, edge_weight, W, b):
    w12 = W[: 2 * D].reshape(2, D)
    b11 = jnp.reshape(b, (1, 1)).astype(jnp.float32)
    pq = _pq_call(h, w12, b11)
    c16 = jnp.broadcast_to(W[2 * D], (L,)).astype(jnp.float32)
    out_flat = _sc_kernel(pq, edge_index[0], edge_index[1],
                          edge_weight, c16)
    return out_flat.reshape(N, N)


# speedup vs baseline: 1.0075x; 1.0075x over previous
import functools

import jax
import jax.numpy as jnp
from jax import lax
from jax.experimental import pallas as pl
from jax.experimental.pallas import tpu as pltpu
from jax.experimental.pallas import tpu_sc as plsc

N = 10000
E = 160000
D = 128

NC = 2
NS = 16
L = 16
NW = NC * NS

EPW = E // NS
GRP = EPW // L
SROW = 1024
CPAD = EPW + SROW + L
FW = (N * N) // NW
FCH = 25000


def _pq_body(h_ref, w_ref, b_ref, pq_ref):
    h = h_ref[...]
    w1 = w_ref[0, :]
    w2 = w_ref[1, :]
    p = jnp.sum(h * w1[None, :], axis=1) + b_ref[0, 0]
    q = jnp.sum(h * w2[None, :], axis=1)
    pq_ref[pl.ds(0, 1), :] = p.reshape(1, N)
    pq_ref[pl.ds(1, 1), :] = q.reshape(1, N)


_pq_call = pl.pallas_call(
    _pq_body,
    out_shape=jax.ShapeDtypeStruct((2, N), jnp.float32),
)


_mesh = plsc.VectorSubcoreMesh(core_axis_name="c", subcore_axis_name="s")


@functools.partial(
    pl.kernel,
    out_type=jax.ShapeDtypeStruct((N * N,), jnp.float32),
    mesh=_mesh,
    scratch_types=[
        pltpu.VMEM((FCH,), jnp.float32),
        pltpu.VMEM((EPW,), jnp.int32),
        pltpu.VMEM((EPW,), jnp.int32),
        pltpu.VMEM((EPW,), jnp.float32),
        pltpu.VMEM((N,), jnp.float32),
        pltpu.VMEM((N,), jnp.float32),
        pltpu.VMEM((L,), jnp.float32),
        pltpu.VMEM((CPAD,), jnp.int32),
        pltpu.VMEM((CPAD,), jnp.float32),
        pltpu.SemaphoreType.DMA,
        pltpu.SemaphoreType.DMA,
    ],
    compiler_params=pltpu.CompilerParams(needs_layout_passes=False),
)
def _sc_kernel(pq_hbm, src_hbm, dst_hbm, w_hbm, c_hbm, out_hbm,
               fill_v, src_v, dst_v, w_v, p_v, q_v, c_v,
               cidx_v, cval_v, sem, fsem):
    cid = lax.axis_index("c")
    sid = lax.axis_index("s")
    wid = cid * NS + sid

    neg = jnp.full((L,), -jnp.inf, jnp.float32)

    def _init_fill(i, carry):
        fill_v[pl.ds(i * L, L)] = neg
        return carry
    lax.fori_loop(0, FCH // L, _init_fill, 0)
    fill_v[pl.ds(FCH - L, L)] = neg

    base = wid * FW

    def _fill_start(k, carry):
        pltpu.async_copy(fill_v, out_hbm.at[pl.ds(base + k * FCH, FCH)], fsem)
        return carry
    lax.fori_loop(0, FW // FCH, _fill_start, 0)

    eoff = sid * EPW
    pltpu.sync_copy(src_hbm.at[pl.ds(eoff, EPW)], src_v)
    pltpu.sync_copy(dst_hbm.at[pl.ds(eoff, EPW)], dst_v)
    pltpu.sync_copy(w_hbm.at[pl.ds(eoff, EPW)], w_v)
    pltpu.sync_copy(pq_hbm.at[0], p_v)
    pltpu.sync_copy(pq_hbm.at[1], q_v)
    pltpu.sync_copy(c_hbm, c_v)

    c_vec = c_v[...]
    lo_v = jnp.broadcast_to(cid * (N // NC), (L,)).astype(jnp.int32)
    hi_v = lo_v + (N // NC)
    n_vec = jnp.full((L,), N, jnp.int32)
    iota = lax.iota(jnp.int32, L)
    np1_vec = jnp.full((L,), N + 1, jnp.int32)

    def _compute(j, cnt):
        e0 = j * L
        sv = src_v[pl.ds(e0, L)]
        dv = dst_v[pl.ds(e0, L)]
        wv = w_v[pl.ds(e0, L)]
        pv = plsc.load_gather(p_v, [sv])
        qv = plsc.load_gather(q_v, [dv])
        sc = pv + qv + wv * c_vec
        valid = (sv >= lo_v) & (sv < hi_v) & (sv != dv)
        plsc.store_compressed(cidx_v.at[pl.ds(cnt, L)], sv * n_vec + dv,
                              mask=valid)
        plsc.store_compressed(cval_v.at[pl.ds(cnt, L)], sc, mask=valid)
        return cnt + jnp.sum(valid.astype(jnp.int32))
    cnt = lax.fori_loop(0, GRP, _compute, jnp.int32(0))

    for t in range(SROW // L):
        cidx_v[pl.ds(cnt + t * L, L)] = (lo_v + t * L + iota) * np1_vec
        cval_v[pl.ds(cnt + t * L, L)] = neg
    nrows = (cnt + SROW - 1) // SROW

    def _fill_wait(k, carry):
        pltpu.make_async_copy(
            fill_v, out_hbm.at[pl.ds(base + k * FCH, FCH)], fsem).wait()
        return carry
    lax.fori_loop(0, FW // FCH, _fill_wait, 0)

    plsc.subcore_barrier()

    def _scatter_start(j, carry):
        pltpu.async_copy(cval_v.at[pl.ds(j * SROW, SROW)],
                         out_hbm.at[cidx_v.at[pl.ds(j * SROW, SROW)]], sem)
        return carry
    lax.fori_loop(0, nrows, _scatter_start, 0)

    def _scatter_wait(j, carry):
        pltpu.make_async_copy(cval_v.at[pl.ds(j * SROW, SROW)],
                              out_hbm.at[cidx_v.at[pl.ds(j * SROW, SROW)]],
                              sem).wait()
        return carry
    lax.fori_loop(0, nrows, _scatter_wait, 0)


def kernel(h, edge_index, edge_weight, W, b):
    w12 = W[: 2 * D].reshape(2, D)
    b11 = jnp.reshape(b, (1, 1)).astype(jnp.float32)
    pq = _pq_call(h, w12, b11)
    c16 = jnp.broadcast_to(W[2 * D], (L,)).astype(jnp.float32)
    out_flat = _sc_kernel(pq, edge_index[0], edge_index[1],
                          edge_weight, c16)
    return out_flat.reshape(N, N)
